# baseline (speedup 1.0000x reference)
import jax
import jax.numpy as jnp
from jax import lax
from jax.experimental import pallas as pl
from jax.experimental.pallas import tpu as pltpu
from jax.experimental.pallas import tpu_sc as plsc

N = 10000
D = 128
E = 320000
NC = 2
NS = 16
NW = NC * NS
C = 96
NCHUNK = (E + C - 1) // C
E_EXT = NCHUNK * C
CPT0 = 153
CPT1 = 55
XTRA = NCHUNK - 16 * (CPT0 + CPT1)
NPAD = 10240
RPT = NPAD // NS
NBUF = 3
NMETA = 6


def _spmm_body(x_hbm, src_hbm, dst_hbm, w_hbm, zero_hbm, acc_out,
               sv, dv, wv, rows, acc_sh,
               msem0, msem1, msem2, msem3, msem4, msem5,
               gsem0, gsem1, gsem2, ssem0, ssem1, ssem2):
    c = lax.axis_index("c")
    s = lax.axis_index("s")

    ncpt = jnp.where(c == 0, CPT0, CPT1 + jnp.where(s < XTRA, 1, 0))
    toff = jnp.where(c == 0, s * CPT0,
                     16 * CPT0 + s * CPT1 + jnp.minimum(s, XTRA))

    msems = (msem0, msem1, msem2, msem3, msem4, msem5)
    gsems = (gsem0, gsem1, gsem2)
    ssems = (ssem0, ssem1, ssem2)

    def issue_meta(k, m):
        off = pl.multiple_of((toff + k) * C, 8)
        pltpu.async_copy(src_hbm.at[pl.ds(off, C)], sv.at[m], msems[m])
        pltpu.async_copy(dst_hbm.at[pl.ds(off, C)], dv.at[m], msems[m])
        pltpu.async_copy(w_hbm.at[pl.ds(off, C)], wv.at[m], msems[m])

    def wait_meta(m):
        pltpu.make_async_copy(src_hbm.at[pl.ds(0, C)], sv.at[m],
                              msems[m]).wait()
        pltpu.make_async_copy(dst_hbm.at[pl.ds(0, C)], dv.at[m],
                              msems[m]).wait()
        pltpu.make_async_copy(w_hbm.at[pl.ds(0, C)], wv.at[m],
                              msems[m]).wait()

    def issue_gather(m, b):
        pltpu.async_copy(x_hbm.at[sv.at[m]], rows.at[b], gsems[b])

    def wait_gather(b):
        pltpu.make_async_copy(x_hbm.at[sv.at[0]], rows.at[b],
                              gsems[b]).wait()

    def issue_scatter(m, b):
        pltpu.async_copy(rows.at[b], acc_sh.at[dv.at[m]], ssems[b],
                         add=True)

    def wait_scatter(b):
        pltpu.make_async_copy(rows.at[b], acc_sh.at[dv.at[0]],
                              ssems[b]).wait()

    def scale(m, b):
        def group_body(g):
            w16 = wv[m, pl.ds(g * 16, 16)]
            for i in range(16):
                ws = w16[i]
                r = g * 16 + i
                for j in range(D // 16):
                    sl = pl.ds(j * 16, 16)
                    rows[b, r, sl] = rows[b, r, sl] * ws
        pl.loop(0, C // 16)(group_body)

    issue_meta(0, 0)
    issue_meta(1, 1)
    pltpu.sync_copy(zero_hbm.at[pl.ds(s * RPT, RPT)],
                    acc_sh.at[pl.ds(s * RPT, RPT)])
    plsc.subcore_barrier()
    wait_meta(0)
    issue_gather(0, 0)

    def t_body(t):
        for q in range(NMETA):
            k = NMETA * t + q
            b = q % NBUF
            m = q
            m1 = (q + 1) % NMETA
            m2 = (q + 2) % NMETA
            b1 = (q + 1) % NBUF

            @pl.when(k < ncpt)
            def _(k=k, b=b, m=m, m1=m1, m2=m2, b1=b1):
                @pl.when(k + 2 < ncpt)
                def _():
                    issue_meta(k + 2, m2)

                @pl.when(k + 1 < ncpt)
                def _():
                    wait_meta(m1)

                    @pl.when(k >= 2)
                    def _():
                        wait_scatter(b1)
                    issue_gather(m1, b1)
                wait_gather(b)
                scale(m, b)
                issue_scatter(m, b)

    pl.loop(0, (ncpt + NMETA - 1) // NMETA)(t_body)
    wait_scatter(0)
    wait_scatter(1)
    wait_scatter(2)

    plsc.subcore_barrier()
    pltpu.sync_copy(acc_sh.at[pl.ds(s * RPT, RPT)],
                    acc_out.at[c, pl.ds(s * RPT, RPT)])


@jax.jit
def _spmm(x, src, dst, w, zeros):
    mesh = plsc.VectorSubcoreMesh(core_axis_name="c", subcore_axis_name="s")
    return pl.kernel(
        _spmm_body,
        out_type=jax.ShapeDtypeStruct((NC, NPAD, D), jnp.float32),
        mesh=mesh,
        scratch_types=[
            pltpu.VMEM((NMETA, C), jnp.int32),
            pltpu.VMEM((NMETA, C), jnp.int32),
            pltpu.VMEM((NMETA, C), jnp.float32),
            pltpu.VMEM((NBUF, C, D), jnp.float32),
            pltpu.VMEM_SHARED((NPAD, D), jnp.float32),
        ] + [pltpu.SemaphoreType.DMA] * (NMETA + 2 * NBUF),
    )(x, src, dst, w, zeros)


def _combine_body(a_ref, b_ref, f_ref, h_out, f_out):
    h = a_ref[0] + b_ref[0]
    h_out[...] = h
    f_out[...] = f_ref[...] + h


@jax.jit
def _combine(acc, f_prev):
    blk = 1000
    grid = N // blk
    spec2 = pl.BlockSpec((blk, D), lambda i: (i, 0))
    return pl.pallas_call(
        _combine_body,
        grid=(grid,),
        in_specs=[pl.BlockSpec((1, blk, D), lambda i: (0, i, 0)),
                  pl.BlockSpec((1, blk, D), lambda i: (1, i, 0)),
                  spec2],
        out_specs=[spec2, spec2],
        out_shape=[jax.ShapeDtypeStruct((N, D), jnp.float32),
                   jax.ShapeDtypeStruct((N, D), jnp.float32)],
    )(acc, acc, f_prev)


def kernel(x, edge_index, edge_weight):
    pad = E_EXT - E
    dst = jnp.concatenate([edge_index[0], jnp.zeros((pad,), jnp.int32)])
    src = jnp.concatenate([edge_index[1], jnp.zeros((pad,), jnp.int32)])
    w = jnp.concatenate([edge_weight, jnp.zeros((pad,), jnp.float32)])
    zeros = jnp.zeros((NPAD, D), jnp.float32)

    acc1 = _spmm(x, src, dst, w, zeros)
    h1, fin1 = _combine(acc1, x)
    acc2 = _spmm(h1, src, dst, w, zeros)
    _, fin = _combine(acc2, fin1)
    return fin


# speedup vs baseline: 1.2816x; 1.2816x over previous
import jax
import jax.numpy as jnp
from jax import lax
from jax.experimental import pallas as pl
from jax.experimental.pallas import tpu as pltpu
from jax.experimental.pallas import tpu_sc as plsc

N = 10000
D = 128
E = 320000
NC = 2
NS = 16
NW = NC * NS
C = 96
NCHUNK = (E + C - 1) // C
E_EXT = NCHUNK * C
CPT0 = 108
CPT1 = 100
XTRA = NCHUNK - 16 * (CPT0 + CPT1)
NPAD = 10240
RPT = NPAD // NS
NBUF = 3
NMETA = 6


def _spmm_body(x_hbm, src_hbm, dst_hbm, w_hbm, zero_hbm, acc_out,
               sv, dv, wv, rows, acc_sh,
               msem0, msem1, msem2, msem3, msem4, msem5,
               gsem0, gsem1, gsem2, ssem0, ssem1, ssem2):
    c = lax.axis_index("c")
    s = lax.axis_index("s")

    ncpt = jnp.where(c == 0, CPT0, CPT1 + jnp.where(s < XTRA, 1, 0))
    toff = jnp.where(c == 0, s * CPT0,
                     16 * CPT0 + s * CPT1 + jnp.minimum(s, XTRA))

    msems = (msem0, msem1, msem2, msem3, msem4, msem5)
    gsems = (gsem0, gsem1, gsem2)
    ssems = (ssem0, ssem1, ssem2)

    def issue_meta(k, m):
        off = pl.multiple_of((toff + k) * C, 8)
        pltpu.async_copy(src_hbm.at[pl.ds(off, C)], sv.at[m], msems[m])
        pltpu.async_copy(dst_hbm.at[pl.ds(off, C)], dv.at[m], msems[m])
        pltpu.async_copy(w_hbm.at[pl.ds(off, C)], wv.at[m], msems[m])

    def wait_meta(m):
        pltpu.make_async_copy(src_hbm.at[pl.ds(0, C)], sv.at[m],
                              msems[m]).wait()
        pltpu.make_async_copy(dst_hbm.at[pl.ds(0, C)], dv.at[m],
                              msems[m]).wait()
        pltpu.make_async_copy(w_hbm.at[pl.ds(0, C)], wv.at[m],
                              msems[m]).wait()

    def issue_gather(m, b):
        pltpu.async_copy(x_hbm.at[sv.at[m]], rows.at[b], gsems[b])

    def wait_gather(b):
        pltpu.make_async_copy(x_hbm.at[sv.at[0]], rows.at[b],
                              gsems[b]).wait()

    def issue_scatter(m, b):
        pltpu.async_copy(rows.at[b], acc_sh.at[dv.at[m]], ssems[b],
                         add=True)

    def wait_scatter(b):
        pltpu.make_async_copy(rows.at[b], acc_sh.at[dv.at[0]],
                              ssems[b]).wait()

    def scale(m, b):
        def group_body(g):
            w16 = wv[m, pl.ds(g * 16, 16)]
            for i in range(16):
                ws = w16[i]
                r = g * 16 + i
                for j in range(D // 16):
                    sl = pl.ds(j * 16, 16)
                    rows[b, r, sl] = rows[b, r, sl] * ws
        pl.loop(0, C // 16)(group_body)

    issue_meta(0, 0)
    issue_meta(1, 1)
    pltpu.sync_copy(zero_hbm.at[pl.ds(s * RPT, RPT)],
                    acc_sh.at[pl.ds(s * RPT, RPT)])
    plsc.subcore_barrier()
    wait_meta(0)
    issue_gather(0, 0)

    def t_body(t):
        for q in range(NMETA):
            k = NMETA * t + q
            b = q % NBUF
            m = q
            m1 = (q + 1) % NMETA
            m2 = (q + 2) % NMETA
            b1 = (q + 1) % NBUF

            @pl.when(k < ncpt)
            def _(k=k, b=b, m=m, m1=m1, m2=m2, b1=b1):
                @pl.when(k + 2 < ncpt)
                def _():
                    issue_meta(k + 2, m2)

                @pl.when(k + 1 < ncpt)
                def _():
                    wait_meta(m1)

                    @pl.when(k >= 2)
                    def _():
                        wait_scatter(b1)
                    issue_gather(m1, b1)
                wait_gather(b)
                scale(m, b)
                issue_scatter(m, b)

    pl.loop(0, (ncpt + NMETA - 1) // NMETA)(t_body)
    wait_scatter(0)
    wait_scatter(1)
    wait_scatter(2)

    plsc.subcore_barrier()
    pltpu.sync_copy(acc_sh.at[pl.ds(s * RPT, RPT)],
                    acc_out.at[c, pl.ds(s * RPT, RPT)])


@jax.jit
def _spmm(x, src, dst, w, zeros):
    mesh = plsc.VectorSubcoreMesh(core_axis_name="c", subcore_axis_name="s")
    return pl.kernel(
        _spmm_body,
        out_type=jax.ShapeDtypeStruct((NC, NPAD, D), jnp.float32),
        mesh=mesh,
        scratch_types=[
            pltpu.VMEM((NMETA, C), jnp.int32),
            pltpu.VMEM((NMETA, C), jnp.int32),
            pltpu.VMEM((NMETA, C), jnp.float32),
            pltpu.VMEM((NBUF, C, D), jnp.float32),
            pltpu.VMEM_SHARED((NPAD, D), jnp.float32),
        ] + [pltpu.SemaphoreType.DMA] * (NMETA + 2 * NBUF),
    )(x, src, dst, w, zeros)


def _combine_body(a_ref, b_ref, f_ref, h_out, f_out):
    h = a_ref[0] + b_ref[0]
    h_out[...] = h
    f_out[...] = f_ref[...] + h


@jax.jit
def _combine(acc, f_prev):
    blk = 1000
    grid = N // blk
    spec2 = pl.BlockSpec((blk, D), lambda i: (i, 0))
    return pl.pallas_call(
        _combine_body,
        grid=(grid,),
        in_specs=[pl.BlockSpec((1, blk, D), lambda i: (0, i, 0)),
                  pl.BlockSpec((1, blk, D), lambda i: (1, i, 0)),
                  spec2],
        out_specs=[spec2, spec2],
        out_shape=[jax.ShapeDtypeStruct((N, D), jnp.float32),
                   jax.ShapeDtypeStruct((N, D), jnp.float32)],
    )(acc, acc, f_prev)


def kernel(x, edge_index, edge_weight):
    pad = E_EXT - E
    dst = jnp.concatenate([edge_index[0], jnp.zeros((pad,), jnp.int32)])
    src = jnp.concatenate([edge_index[1], jnp.zeros((pad,), jnp.int32)])
    w = jnp.concatenate([edge_weight, jnp.zeros((pad,), jnp.float32)])
    zeros = jnp.zeros((NPAD, D), jnp.float32)

    acc1 = _spmm(x, src, dst, w, zeros)
    h1, fin1 = _combine(acc1, x)
    acc2 = _spmm(h1, src, dst, w, zeros)
    _, fin = _combine(acc2, fin1)
    return fin


# speedup vs baseline: 1.3112x; 1.0231x over previous
import jax
import jax.numpy as jnp
from jax import lax
from jax.experimental import pallas as pl
from jax.experimental.pallas import tpu as pltpu
from jax.experimental.pallas import tpu_sc as plsc

N = 10000
D = 128
E = 320000
NC = 2
NS = 16
NW = NC * NS
C = 96
NCHUNK = (E + C - 1) // C
E_EXT = NCHUNK * C
CPT0 = 105
CPT1 = 103
XTRA = NCHUNK - 16 * (CPT0 + CPT1)
NPAD = 10240
RPT = NPAD // NS
NBUF = 3
NMETA = 6


def _spmm_body(x_hbm, src_hbm, dst_hbm, w_hbm, zero_hbm, acc_out,
               sv, dv, wv, rows, acc_sh,
               msem0, msem1, msem2, msem3, msem4, msem5,
               gsem0, gsem1, gsem2, ssem0, ssem1, ssem2):
    c = lax.axis_index("c")
    s = lax.axis_index("s")

    ncpt = jnp.where(c == 0, CPT0, CPT1 + jnp.where(s < XTRA, 1, 0))
    toff = jnp.where(c == 0, s * CPT0,
                     16 * CPT0 + s * CPT1 + jnp.minimum(s, XTRA))

    msems = (msem0, msem1, msem2, msem3, msem4, msem5)
    gsems = (gsem0, gsem1, gsem2)
    ssems = (ssem0, ssem1, ssem2)

    def issue_meta(k, m):
        off = pl.multiple_of((toff + k) * C, 8)
        pltpu.async_copy(src_hbm.at[pl.ds(off, C)], sv.at[m], msems[m])
        pltpu.async_copy(dst_hbm.at[pl.ds(off, C)], dv.at[m], msems[m])
        pltpu.async_copy(w_hbm.at[pl.ds(off, C)], wv.at[m], msems[m])

    def wait_meta(m):
        pltpu.make_async_copy(src_hbm.at[pl.ds(0, C)], sv.at[m],
                              msems[m]).wait()
        pltpu.make_async_copy(dst_hbm.at[pl.ds(0, C)], dv.at[m],
                              msems[m]).wait()
        pltpu.make_async_copy(w_hbm.at[pl.ds(0, C)], wv.at[m],
                              msems[m]).wait()

    def issue_gather(m, b):
        pltpu.async_copy(x_hbm.at[sv.at[m]], rows.at[b], gsems[b])

    def wait_gather(b):
        pltpu.make_async_copy(x_hbm.at[sv.at[0]], rows.at[b],
                              gsems[b]).wait()

    def issue_scatter(m, b):
        pltpu.async_copy(rows.at[b], acc_sh.at[dv.at[m]], ssems[b],
                         add=True)

    def wait_scatter(b):
        pltpu.make_async_copy(rows.at[b], acc_sh.at[dv.at[0]],
                              ssems[b]).wait()

    def scale(m, b):
        def group_body(g):
            w16 = wv[m, pl.ds(g * 16, 16)]
            for i in range(16):
                ws = w16[i]
                r = g * 16 + i
                for j in range(D // 16):
                    sl = pl.ds(j * 16, 16)
                    rows[b, r, sl] = rows[b, r, sl] * ws
        pl.loop(0, C // 16)(group_body)

    issue_meta(0, 0)
    issue_meta(1, 1)
    pltpu.sync_copy(zero_hbm.at[pl.ds(s * RPT, RPT)],
                    acc_sh.at[pl.ds(s * RPT, RPT)])
    plsc.subcore_barrier()
    wait_meta(0)
    issue_gather(0, 0)

    def t_body(t):
        for q in range(NMETA):
            k = NMETA * t + q
            b = q % NBUF
            m = q
            m1 = (q + 1) % NMETA
            m2 = (q + 2) % NMETA
            b1 = (q + 1) % NBUF

            @pl.when(k < ncpt)
            def _(k=k, b=b, m=m, m1=m1, m2=m2, b1=b1):
                @pl.when(k + 2 < ncpt)
                def _():
                    issue_meta(k + 2, m2)

                @pl.when(k + 1 < ncpt)
                def _():
                    wait_meta(m1)

                    @pl.when(k >= 2)
                    def _():
                        wait_scatter(b1)
                    issue_gather(m1, b1)
                wait_gather(b)
                scale(m, b)
                issue_scatter(m, b)

    pl.loop(0, (ncpt + NMETA - 1) // NMETA)(t_body)
    wait_scatter(0)
    wait_scatter(1)
    wait_scatter(2)

    plsc.subcore_barrier()
    pltpu.sync_copy(acc_sh.at[pl.ds(s * RPT, RPT)],
                    acc_out.at[c, pl.ds(s * RPT, RPT)])


@jax.jit
def _spmm(x, src, dst, w, zeros):
    mesh = plsc.VectorSubcoreMesh(core_axis_name="c", subcore_axis_name="s")
    return pl.kernel(
        _spmm_body,
        out_type=jax.ShapeDtypeStruct((NC, NPAD, D), jnp.float32),
        mesh=mesh,
        scratch_types=[
            pltpu.VMEM((NMETA, C), jnp.int32),
            pltpu.VMEM((NMETA, C), jnp.int32),
            pltpu.VMEM((NMETA, C), jnp.float32),
            pltpu.VMEM((NBUF, C, D), jnp.float32),
            pltpu.VMEM_SHARED((NPAD, D), jnp.float32),
        ] + [pltpu.SemaphoreType.DMA] * (NMETA + 2 * NBUF),
    )(x, src, dst, w, zeros)


def _combine_body(a_ref, b_ref, f_ref, h_out, f_out):
    h = a_ref[0] + b_ref[0]
    h_out[...] = h
    f_out[...] = f_ref[...] + h


@jax.jit
def _combine(acc, f_prev):
    blk = 1000
    grid = N // blk
    spec2 = pl.BlockSpec((blk, D), lambda i: (i, 0))
    return pl.pallas_call(
        _combine_body,
        grid=(grid,),
        in_specs=[pl.BlockSpec((1, blk, D), lambda i: (0, i, 0)),
                  pl.BlockSpec((1, blk, D), lambda i: (1, i, 0)),
                  spec2],
        out_specs=[spec2, spec2],
        out_shape=[jax.ShapeDtypeStruct((N, D), jnp.float32),
                   jax.ShapeDtypeStruct((N, D), jnp.float32)],
    )(acc, acc, f_prev)


def kernel(x, edge_index, edge_weight):
    pad = E_EXT - E
    dst = jnp.concatenate([edge_index[0], jnp.zeros((pad,), jnp.int32)])
    src = jnp.concatenate([edge_index[1], jnp.zeros((pad,), jnp.int32)])
    w = jnp.concatenate([edge_weight, jnp.zeros((pad,), jnp.float32)])
    zeros = jnp.zeros((NPAD, D), jnp.float32)

    acc1 = _spmm(x, src, dst, w, zeros)
    h1, fin1 = _combine(acc1, x)
    acc2 = _spmm(h1, src, dst, w, zeros)
    _, fin = _combine(acc2, fin1)
    return fin


# speedup vs baseline: 1.3251x; 1.0106x over previous
import jax
import jax.numpy as jnp
from jax import lax
from jax.experimental import pallas as pl
from jax.experimental.pallas import tpu as pltpu
from jax.experimental.pallas import tpu_sc as plsc

N = 10000
D = 128
E = 320000
NC = 2
NS = 16
NW = NC * NS
C = 96
NCHUNK = (E + C - 1) // C
E_EXT = NCHUNK * C
CPT0 = 104
CPT1 = 104
XTRA = NCHUNK - 16 * (CPT0 + CPT1)
NPAD = 10240
RPT = NPAD // NS
NBUF = 3
NMETA = 6


def _spmm_body(x_hbm, src_hbm, dst_hbm, w_hbm, zero_hbm, acc_out,
               sv, dv, wv, rows, acc_sh,
               msem0, msem1, msem2, msem3, msem4, msem5,
               gsem0, gsem1, gsem2, ssem0, ssem1, ssem2):
    c = lax.axis_index("c")
    s = lax.axis_index("s")

    ncpt = jnp.where(c == 0, CPT0, CPT1 + jnp.where(s < XTRA, 1, 0))
    toff = jnp.where(c == 0, s * CPT0,
                     16 * CPT0 + s * CPT1 + jnp.minimum(s, XTRA))

    msems = (msem0, msem1, msem2, msem3, msem4, msem5)
    gsems = (gsem0, gsem1, gsem2)
    ssems = (ssem0, ssem1, ssem2)

    def issue_meta(k, m):
        off = pl.multiple_of((toff + k) * C, 8)
        pltpu.async_copy(src_hbm.at[pl.ds(off, C)], sv.at[m], msems[m])
        pltpu.async_copy(dst_hbm.at[pl.ds(off, C)], dv.at[m], msems[m])
        pltpu.async_copy(w_hbm.at[pl.ds(off, C)], wv.at[m], msems[m])

    def wait_meta(m):
        pltpu.make_async_copy(src_hbm.at[pl.ds(0, C)], sv.at[m],
                              msems[m]).wait()
        pltpu.make_async_copy(dst_hbm.at[pl.ds(0, C)], dv.at[m],
                              msems[m]).wait()
        pltpu.make_async_copy(w_hbm.at[pl.ds(0, C)], wv.at[m],
                              msems[m]).wait()

    def issue_gather(m, b):
        pltpu.async_copy(x_hbm.at[sv.at[m]], rows.at[b], gsems[b])

    def wait_gather(b):
        pltpu.make_async_copy(x_hbm.at[sv.at[0]], rows.at[b],
                              gsems[b]).wait()

    def issue_scatter(m, b):
        pltpu.async_copy(rows.at[b], acc_sh.at[dv.at[m]], ssems[b],
                         add=True)

    def wait_scatter(b):
        pltpu.make_async_copy(rows.at[b], acc_sh.at[dv.at[0]],
                              ssems[b]).wait()

    def scale(m, b):
        def group_body(g):
            w16 = wv[m, pl.ds(g * 16, 16)]
            for i in range(16):
                ws = w16[i]
                r = g * 16 + i
                for j in range(D // 16):
                    sl = pl.ds(j * 16, 16)
                    rows[b, r, sl] = rows[b, r, sl] * ws
        pl.loop(0, C // 16)(group_body)

    issue_meta(0, 0)
    issue_meta(1, 1)
    pltpu.sync_copy(zero_hbm.at[pl.ds(s * RPT, RPT)],
                    acc_sh.at[pl.ds(s * RPT, RPT)])
    plsc.subcore_barrier()
    wait_meta(0)
    issue_gather(0, 0)

    def t_body(t):
        for q in range(NMETA):
            k = NMETA * t + q
            b = q % NBUF
            m = q
            m1 = (q + 1) % NMETA
            m2 = (q + 2) % NMETA
            b1 = (q + 1) % NBUF

            @pl.when(k < ncpt)
            def _(k=k, b=b, m=m, m1=m1, m2=m2, b1=b1):
                @pl.when(k + 2 < ncpt)
                def _():
                    issue_meta(k + 2, m2)

                @pl.when(k + 1 < ncpt)
                def _():
                    wait_meta(m1)

                    @pl.when(k >= 2)
                    def _():
                        wait_scatter(b1)
                    issue_gather(m1, b1)
                wait_gather(b)
                scale(m, b)
                issue_scatter(m, b)

    pl.loop(0, (ncpt + NMETA - 1) // NMETA)(t_body)
    wait_scatter(0)
    wait_scatter(1)
    wait_scatter(2)

    plsc.subcore_barrier()
    pltpu.sync_copy(acc_sh.at[pl.ds(s * RPT, RPT)],
                    acc_out.at[c, pl.ds(s * RPT, RPT)])


@jax.jit
def _spmm(x, src, dst, w, zeros):
    mesh = plsc.VectorSubcoreMesh(core_axis_name="c", subcore_axis_name="s")
    return pl.kernel(
        _spmm_body,
        out_type=jax.ShapeDtypeStruct((NC, NPAD, D), jnp.float32),
        mesh=mesh,
        scratch_types=[
            pltpu.VMEM((NMETA, C), jnp.int32),
            pltpu.VMEM((NMETA, C), jnp.int32),
            pltpu.VMEM((NMETA, C), jnp.float32),
            pltpu.VMEM((NBUF, C, D), jnp.float32),
            pltpu.VMEM_SHARED((NPAD, D), jnp.float32),
        ] + [pltpu.SemaphoreType.DMA] * (NMETA + 2 * NBUF),
    )(x, src, dst, w, zeros)


def _combine_body(a_ref, b_ref, f_ref, h_out, f_out):
    h = a_ref[0] + b_ref[0]
    h_out[...] = h
    f_out[...] = f_ref[...] + h


@jax.jit
def _combine(acc, f_prev):
    blk = 1000
    grid = N // blk
    spec2 = pl.BlockSpec((blk, D), lambda i: (i, 0))
    return pl.pallas_call(
        _combine_body,
        grid=(grid,),
        in_specs=[pl.BlockSpec((1, blk, D), lambda i: (0, i, 0)),
                  pl.BlockSpec((1, blk, D), lambda i: (1, i, 0)),
                  spec2],
        out_specs=[spec2, spec2],
        out_shape=[jax.ShapeDtypeStruct((N, D), jnp.float32),
                   jax.ShapeDtypeStruct((N, D), jnp.float32)],
    )(acc, acc, f_prev)


def kernel(x, edge_index, edge_weight):
    pad = E_EXT - E
    dst = jnp.concatenate([edge_index[0], jnp.zeros((pad,), jnp.int32)])
    src = jnp.concatenate([edge_index[1], jnp.zeros((pad,), jnp.int32)])
    w = jnp.concatenate([edge_weight, jnp.zeros((pad,), jnp.float32)])
    zeros = jnp.zeros((NPAD, D), jnp.float32)

    acc1 = _spmm(x, src, dst, w, zeros)
    h1, fin1 = _combine(acc1, x)
    acc2 = _spmm(h1, src, dst, w, zeros)
    _, fin = _combine(acc2, fin1)
    return fin
